# baseline (speedup 1.0000x reference)
import jax
import jax.numpy as jnp
import numpy as np
from jax.experimental import pallas as pl
from jax.experimental.pallas import tpu as pltpu

D = 14
TREE = 2 ** D - 1
NT = 6
XS = 128
HS = 128
LEAF0 = 2 ** (D - 1) - 1
LEAVES = 2 ** (D - 1)
DBITS = 7
ABITS = D - 1 - DBITS
CHUNK_LEVELS = DBITS
STOP_ROWS = LEAVES >> CHUNK_LEVELS
MID_ROWS = NT * STOP_ROWS
TOP_LEVELS = D - 1 - CHUNK_LEVELS
LEAF_SPLIT = 8


def _pair_matrix(rows):
    r = np.arange(rows)[:, None]
    c = np.arange(2 * rows)[None, :]
    return jnp.asarray((c // 2 == r).astype(np.float32))


def _sig(x):
    return 0.5 * jnp.tanh(0.5 * x) + 0.5


def _sig_pre(t):
    return t * 0.5 + 0.5


def _gates(iou, c_acc):
    i = _sig(iou[:, :HS])
    o = _sig(iou[:, HS:2 * HS])
    u = jnp.tanh(iou[:, 2 * HS:])
    c = i * u if c_acc is None else i * u + c_acc
    h = o * jnp.tanh(c)
    return h, c


def _gates_pre(iou, c_acc):
    i = _sig_pre(jnp.tanh(iou[:, :HS]))
    o = _sig_pre(jnp.tanh(iou[:, HS:2 * HS]))
    u = jnp.tanh(iou[:, 2 * HS:])
    c = i * u if c_acc is None else i * u + c_acc
    h = o * jnp.tanh(c)
    return h, c


def _mm_t(x, w):
    return jax.lax.dot_general(x, w, (((1,), (1,)), ((), ())),
                               preferred_element_type=jnp.float32)


def _rev7(d):
    r = 0
    for i in range(7):
        r |= ((d >> i) & 1) << (6 - i)
    return r


def _tree_copies(x_hbm, xbuf, sem, tree, slot):
    out = []
    for d in range(2 ** DBITS):
        out.append(pltpu.make_async_copy(
            x_hbm.at[tree, :, pl.ds(d * XS, XS)],
            xbuf.at[slot, pl.ds(_rev7(d) * (2 ** ABITS), 2 ** ABITS), :],
            sem.at[slot]))
    return out


def _leaf_pair(x_ref, w_ref, b_ref, ufw_ref, ufb_ref, sl):
    xs = x_ref[sl, :]
    iou = _mm_t(xs, w_ref[...]) + b_ref[...]
    h, c = _gates_pre(iou, None)
    f = _sig_pre(jnp.tanh(_mm_t(h.astype(jnp.bfloat16), ufw_ref[...])
                          + ufb_ref[...]))
    return h, f * c


def _bu_tree_kernel(x_hbm, w_ref, u_ref, b_ref, ufw_ref, ufb_ref,
                    mh_ref, mc_ref, xbuf, sem):
    t = pl.program_id(0)

    @pl.when(t == 0)
    def _():
        for cp in _tree_copies(x_hbm, xbuf, sem, t, 0):
            cp.start()

    @pl.when(t + 1 < NT)
    def _():
        for cp in _tree_copies(x_hbm, xbuf, sem, t + 1, (t + 1) % 2):
            cp.start()

    slot = t % 2
    for cp in _tree_copies(x_hbm, xbuf, sem, t, slot):
        cp.wait()
    x_ref = xbuf.at[slot]

    hs, cs = [], []
    step = LEAVES // LEAF_SPLIT
    npair = LEAF_SPLIT // 2
    for s in range(npair):
        h1, fc1 = _leaf_pair(x_ref, w_ref, b_ref, ufw_ref, ufb_ref,
                             slice(s * step, (s + 1) * step))
        h2, fc2 = _leaf_pair(x_ref, w_ref, b_ref, ufw_ref, ufb_ref,
                             slice((s + npair) * step, (s + npair + 1) * step))
        iou = _mm_t((h1 + h2).astype(jnp.bfloat16), u_ref[...]) + b_ref[...]
        h_s, c_s = _gates_pre(iou, fc1 + fc2)
        hs.append(h_s)
        cs.append(c_s)
    h = jnp.concatenate(hs, axis=0)
    c = jnp.concatenate(cs, axis=0)
    rows = LEAVES // 2
    for _ in range(CHUNK_LEVELS - 1):
        f = _sig_pre(jnp.tanh(_mm_t(h.astype(jnp.bfloat16), ufw_ref[...])
                              + ufb_ref[...]))
        fc = f * c
        half = rows // 2
        c_acc = fc[:half] + fc[half:]
        h_tild = h[:half] + h[half:]
        iou = _mm_t(h_tild.astype(jnp.bfloat16), u_ref[...]) + b_ref[...]
        h, c = _gates_pre(iou, c_acc)
        rows = half
    mh_ref[...] = h
    mc_ref[...] = c


def _top_kernel(mh_ref, mc_ref, p5_ref, p4_ref, p3_ref, p2_ref, p1_ref,
                p0_ref, u_ref, b_ref, ufw_ref, ufb_ref, xroot_ref, wtd_ref,
                utdc_ref, btd_ref, uftdb_ref, out_ref):
    h = mh_ref[...]
    c = mc_ref[...]
    for p_ref in (p5_ref, p4_ref, p3_ref, p2_ref, p1_ref, p0_ref):
        f = _sig(_mm_t(h, ufw_ref[...]) + ufb_ref[...])
        p = p_ref[...]
        c_acc = jnp.dot(p, f * c, preferred_element_type=jnp.float32)
        h_tild = jnp.dot(p, h, preferred_element_type=jnp.float32)
        iou = _mm_t(h_tild, u_ref[...]) + b_ref[...]
        h, c = _gates(iou, c_acc)
    out_ref[:, :HS] = h

    iou = (_mm_t(xroot_ref[...], wtd_ref[:, :XS])
           + _mm_t(h, wtd_ref[:, XS:]) + btd_ref[...])
    ht, ct = _gates(iou, None)
    for _ in range(D - 1):
        z = _mm_t(ht, utdc_ref[...])
        f = _sig(z[:, :HS] + uftdb_ref[...])
        iou = z[:, HS:] + btd_ref[...]
        ht, ct = _gates(iou, f * ct)
    out_ref[:, HS:] = ht


def kernel(X, h0, c0, W_iou_bu, U_iou_bu, b_iou_bu, U_f_bu_W, U_f_bu_b,
           W_iou_td, U_iou_td, b_iou_td, U_f_td_W, U_f_td_b):
    Xr = X.reshape(NT, TREE, XS)
    x_leaf = Xr[:, LEAF0:, :].astype(jnp.bfloat16)
    x_leaf = x_leaf.reshape(NT, 2 ** ABITS, 2 ** DBITS * XS)
    x_root = Xr[:, 0, :]
    ufb_bu = U_f_bu_b.reshape(1, HS)
    ufb_td = U_f_td_b.reshape(1, HS)
    utd_comb = jnp.concatenate([U_f_td_W, U_iou_td], axis=0)
    io_scale = jnp.concatenate([jnp.full((2 * HS, 1), 0.5, jnp.float32),
                                jnp.ones((HS, 1), jnp.float32)], axis=0)
    w_bf = (W_iou_bu * io_scale).astype(jnp.bfloat16)
    u_bf = (U_iou_bu * io_scale).astype(jnp.bfloat16)
    b_sc = b_iou_bu * io_scale.T
    ufw_bf = (0.5 * U_f_bu_W).astype(jnp.bfloat16)
    ufb_sc = 0.5 * ufb_bu

    full = lambda shape: pl.BlockSpec(shape, lambda i: tuple(0 for _ in shape))
    mid_h, mid_c = pl.pallas_call(
        _bu_tree_kernel,
        grid=(NT,),
        in_specs=[
            pl.BlockSpec(memory_space=pl.ANY),
            full((3 * HS, XS)),
            full((3 * HS, HS)),
            full((1, 3 * HS)),
            full((HS, HS)),
            full((1, HS)),
        ],
        out_specs=[
            pl.BlockSpec((STOP_ROWS, HS), lambda i: (i, 0)),
            pl.BlockSpec((STOP_ROWS, HS), lambda i: (i, 0)),
        ],
        out_shape=[
            jax.ShapeDtypeStruct((MID_ROWS, HS), jnp.float32),
            jax.ShapeDtypeStruct((MID_ROWS, HS), jnp.float32),
        ],
        scratch_shapes=[
            pltpu.VMEM((2, LEAVES, XS), jnp.bfloat16),
            pltpu.SemaphoreType.DMA((2,)),
        ],
        compiler_params=pltpu.CompilerParams(
            dimension_semantics=("arbitrary",)),
    )(x_leaf, w_bf, u_bf, b_sc, ufw_bf, ufb_sc)

    pmats = [_pair_matrix(NT * (2 ** l)) for l in range(TOP_LEVELS - 1, -1, -1)]
    out = pl.pallas_call(
        _top_kernel,
        out_shape=jax.ShapeDtypeStruct((NT, 2 * HS), jnp.float32),
    )(mid_h, mid_c, *pmats,
      U_iou_bu, b_iou_bu, U_f_bu_W, ufb_bu,
      x_root, W_iou_td, utd_comb, b_iou_td, ufb_td)
    return out


# speedup vs baseline: 1.0466x; 1.0466x over previous
import jax
import jax.numpy as jnp
import numpy as np
from jax.experimental import pallas as pl
from jax.experimental.pallas import tpu as pltpu

D = 14
TREE = 2 ** D - 1
NT = 6
XS = 128
HS = 128
LEAF0 = 2 ** (D - 1) - 1
LEAVES = 2 ** (D - 1)
DBITS = 7
ABITS = D - 1 - DBITS
CHUNK_LEVELS = DBITS
STOP_ROWS = LEAVES >> CHUNK_LEVELS
MID_ROWS = NT * STOP_ROWS
TOP_LEVELS = D - 1 - CHUNK_LEVELS
LEAF_SPLIT = 8


def _pair_matrix(rows):
    r = np.arange(rows)[:, None]
    c = np.arange(2 * rows)[None, :]
    return jnp.asarray((c // 2 == r).astype(np.float32))


def _sig(x):
    return 0.5 * jnp.tanh(0.5 * x) + 0.5


def _sig_pre(t):
    return t * 0.5 + 0.5


def _gates(iou, c_acc):
    i = _sig(iou[:, :HS])
    o = _sig(iou[:, HS:2 * HS])
    u = jnp.tanh(iou[:, 2 * HS:])
    c = i * u if c_acc is None else i * u + c_acc
    h = o * jnp.tanh(c)
    return h, c


def _gates_pre(iou, c_acc):
    i = _sig_pre(jnp.tanh(iou[:, :HS]))
    o = _sig_pre(jnp.tanh(iou[:, HS:2 * HS]))
    u = jnp.tanh(iou[:, 2 * HS:])
    c = i * u if c_acc is None else i * u + c_acc
    h = o * jnp.tanh(c)
    return h, c


def _mm_t(x, w):
    return jax.lax.dot_general(x, w, (((1,), (1,)), ((), ())),
                               preferred_element_type=jnp.float32)


def _rev7(d):
    r = 0
    for i in range(7):
        r |= ((d >> i) & 1) << (6 - i)
    return r


def _tree_copies(x_hbm, xbuf, sem, tree, slot):
    out = []
    for d in range(2 ** DBITS):
        out.append(pltpu.make_async_copy(
            x_hbm.at[tree, :, pl.ds(d * XS, XS)],
            xbuf.at[slot, pl.ds(_rev7(d) * (2 ** ABITS), 2 ** ABITS), :],
            sem.at[slot]))
    return out


def _leaf_pair(x_ref, w_ref, b_ref, ufw_ref, ufb_ref, sl):
    xs = x_ref[sl, :]
    iou = _mm_t(xs, w_ref[...]) + b_ref[...]
    h, c = _gates_pre(iou, None)
    f = _sig_pre(jnp.tanh(_mm_t(h.astype(jnp.bfloat16), ufw_ref[...])
                          + ufb_ref[...]))
    return h, f * c


def _bu_tree_kernel(x_hbm, w_ref, u_ref, b_ref, ufw_ref, ufb_ref,
                    mh_ref, mc_ref, xbuf, sem):
    t = pl.program_id(0)

    @pl.when(t == 0)
    def _():
        for cp in _tree_copies(x_hbm, xbuf, sem, t, 0):
            cp.start()

    @pl.when(t + 1 < NT)
    def _():
        for cp in _tree_copies(x_hbm, xbuf, sem, t + 1, (t + 1) % 2):
            cp.start()

    slot = t % 2
    for cp in _tree_copies(x_hbm, xbuf, sem, t, slot):
        cp.wait()
    x_ref = xbuf.at[slot]

    hs, cs = [], []
    step = LEAVES // LEAF_SPLIT
    npair = LEAF_SPLIT // 2
    for s in range(npair):
        h1, fc1 = _leaf_pair(x_ref, w_ref, b_ref, ufw_ref, ufb_ref,
                             slice(s * step, (s + 1) * step))
        h2, fc2 = _leaf_pair(x_ref, w_ref, b_ref, ufw_ref, ufb_ref,
                             slice((s + npair) * step, (s + npair + 1) * step))
        iou = _mm_t((h1 + h2).astype(jnp.bfloat16), u_ref[...]) + b_ref[...]
        h_s, c_s = _gates_pre(iou, fc1 + fc2)
        hs.append(h_s)
        cs.append(c_s)
    h = jnp.concatenate(hs, axis=0)
    c = jnp.concatenate(cs, axis=0)
    rows = LEAVES // 2
    for _ in range(CHUNK_LEVELS - 1):
        f = _sig_pre(jnp.tanh(_mm_t(h.astype(jnp.bfloat16), ufw_ref[...])
                              + ufb_ref[...]))
        fc = f * c
        half = rows // 2
        c_acc = fc[:half] + fc[half:]
        h_tild = h[:half] + h[half:]
        iou = _mm_t(h_tild.astype(jnp.bfloat16), u_ref[...]) + b_ref[...]
        h, c = _gates_pre(iou, c_acc)
        rows = half
    mh_ref[...] = h
    mc_ref[...] = c


def _top_kernel(mh_ref, mc_ref, p5_ref, p4_ref, p3_ref, p2_ref, p1_ref,
                p0_ref, u_ref, b_ref, ufw_ref, ufb_ref, xroot_ref, wtd_ref,
                utdc_ref, btd_ref, uftdb_ref, out_ref):
    h = mh_ref[...]
    c = mc_ref[...]
    for p_ref in (p5_ref, p4_ref, p3_ref, p2_ref, p1_ref, p0_ref):
        f = _sig(_mm_t(h, ufw_ref[...]) + ufb_ref[...])
        p = p_ref[...]
        c_acc = jnp.dot(p, f * c, preferred_element_type=jnp.float32)
        h_tild = jnp.dot(p, h, preferred_element_type=jnp.float32)
        iou = _mm_t(h_tild, u_ref[...]) + b_ref[...]
        h, c = _gates(iou, c_acc)
    out_ref[:, :HS] = h

    iou = (_mm_t(xroot_ref[...], wtd_ref[:, :XS])
           + _mm_t(h, wtd_ref[:, XS:]) + btd_ref[...])
    ht, ct = _gates(iou, None)
    for _ in range(D - 1):
        z = _mm_t(ht, utdc_ref[...])
        f = _sig(z[:, :HS] + uftdb_ref[...])
        iou = z[:, HS:] + btd_ref[...]
        ht, ct = _gates(iou, f * ct)
    out_ref[:, HS:] = ht


def kernel(X, h0, c0, W_iou_bu, U_iou_bu, b_iou_bu, U_f_bu_W, U_f_bu_b,
           W_iou_td, U_iou_td, b_iou_td, U_f_td_W, U_f_td_b):
    x_leaf = jnp.stack([
        jax.lax.slice(X, (t * TREE + LEAF0, 0), (t * TREE + TREE, XS))
        for t in range(NT)]).astype(jnp.bfloat16)
    x_leaf = x_leaf.reshape(NT, 2 ** ABITS, 2 ** DBITS * XS)
    x_root = jnp.stack([X[t * TREE] for t in range(NT)])
    ufb_bu = U_f_bu_b.reshape(1, HS)
    ufb_td = U_f_td_b.reshape(1, HS)
    utd_comb = jnp.concatenate([U_f_td_W, U_iou_td], axis=0)
    io_scale = jnp.concatenate([jnp.full((2 * HS, 1), 0.5, jnp.float32),
                                jnp.ones((HS, 1), jnp.float32)], axis=0)
    w_bf = (W_iou_bu * io_scale).astype(jnp.bfloat16)
    u_bf = (U_iou_bu * io_scale).astype(jnp.bfloat16)
    b_sc = b_iou_bu * io_scale.T
    ufw_bf = (0.5 * U_f_bu_W).astype(jnp.bfloat16)
    ufb_sc = 0.5 * ufb_bu

    full = lambda shape: pl.BlockSpec(shape, lambda i: tuple(0 for _ in shape))
    mid_h, mid_c = pl.pallas_call(
        _bu_tree_kernel,
        grid=(NT,),
        in_specs=[
            pl.BlockSpec(memory_space=pl.ANY),
            full((3 * HS, XS)),
            full((3 * HS, HS)),
            full((1, 3 * HS)),
            full((HS, HS)),
            full((1, HS)),
        ],
        out_specs=[
            pl.BlockSpec((STOP_ROWS, HS), lambda i: (i, 0)),
            pl.BlockSpec((STOP_ROWS, HS), lambda i: (i, 0)),
        ],
        out_shape=[
            jax.ShapeDtypeStruct((MID_ROWS, HS), jnp.float32),
            jax.ShapeDtypeStruct((MID_ROWS, HS), jnp.float32),
        ],
        scratch_shapes=[
            pltpu.VMEM((2, LEAVES, XS), jnp.bfloat16),
            pltpu.SemaphoreType.DMA((2,)),
        ],
        compiler_params=pltpu.CompilerParams(
            dimension_semantics=("arbitrary",)),
    )(x_leaf, w_bf, u_bf, b_sc, ufw_bf, ufb_sc)

    pmats = [_pair_matrix(NT * (2 ** l)) for l in range(TOP_LEVELS - 1, -1, -1)]
    out = pl.pallas_call(
        _top_kernel,
        out_shape=jax.ShapeDtypeStruct((NT, 2 * HS), jnp.float32),
    )(mid_h, mid_c, *pmats,
      U_iou_bu, b_iou_bu, U_f_bu_W, ufb_bu,
      x_root, W_iou_td, utd_comb, b_iou_td, ufb_td)
    return out
